# baseline (speedup 1.0000x reference)
import jax
import jax.numpy as jnp
from jax import lax
from jax.experimental import pallas as pl
from jax.experimental.pallas import tpu as pltpu
from jax.experimental.pallas import tpu_sc as plsc

N_GRAPHS = 512
N_NODES = 100000
N_EDGES = 1600000
NODE_DIM = 128
EDGE_DIM = 16
HIDDEN = 64

NW = 32
L = 128

N_FULL = N_NODES // L
N_TAIL = N_NODES - N_FULL * L
N_CNT = N_FULL // NW
N_EXTRA = N_FULL - N_CNT * NW
N_MAX = N_CNT + 1

CW = 2048
E_LIMIT = N_EDGES - CW
G_PER_TILE = N_GRAPHS // NW


def _node_phase(src_hbm, idxb, acc, buf0, buf1, sem0, sem1, start, cnt):
    @pl.when(cnt > 0)
    def _():
        pltpu.async_copy(src_hbm.at[pl.ds(start * L, L), :], buf0, sem0)

    @pl.when(cnt > 1)
    def _():
        pltpu.async_copy(src_hbm.at[pl.ds((start + 1) * L, L), :], buf1, sem1)

    def _pair(tp, _):
        for half, (buf, sem) in enumerate(((buf0, sem0), (buf1, sem1))):
            t = 2 * tp + half

            @pl.when(t < cnt)
            def _():
                pltpu.make_async_copy(
                    src_hbm.at[pl.ds((start + t) * L, L), :], buf, sem).wait()
                pltpu.sync_copy(buf, acc.at[idxb.at[pl.ds(t * L, L)]],
                                add=True)

                @pl.when(t + 2 < cnt)
                def _():
                    pltpu.async_copy(
                        src_hbm.at[pl.ds((start + t + 2) * L, L), :], buf, sem)
        return _

    lax.fori_loop(0, (N_MAX + 1) // 2, _pair, None)


def _edge_phase(eT_hbm, starts, ep_out, ebuf0, ebuf1, fbufh, fbuft, obuf,
                esem0, esem1, wid):
    iota = lax.iota(jnp.int32, 16)
    zero16 = jnp.zeros((16,), jnp.float32)
    UNROLL = 8

    def _graph(gloc, _):
        g = wid * G_PER_TILE + gloc
        win = starts[pl.ds(g, 16)]
        a = win[0]
        b = win[1]
        fl_a = pl.multiple_of((a // 16) * 16, 16)
        fl_b = pl.multiple_of((b // 16) * 16, 16)
        ca = pl.multiple_of(jnp.where(a == fl_a, a, fl_a + 16), 16)

        @pl.when(a < b)
        def _():
            pltpu.sync_copy(eT_hbm.at[:, pl.ds(fl_a, 16)], fbufh)

        @pl.when((fl_b > fl_a) & (b != fl_b))
        def _():
            pltpu.sync_copy(eT_hbm.at[:, pl.ds(fl_b, 16)], fbuft)

        head_mask = (iota >= a - fl_a) & (iota < jnp.minimum(ca, b) - fl_a)
        tail_mask = (iota < jnp.where(fl_b > fl_a, b - fl_b, 0))

        nfull = jnp.maximum(fl_b - ca, 0)
        n_chunks = (nfull + CW - 1) // CW

        def _cstart(t):
            d_k = ca + t * CW
            return pl.multiple_of(jnp.minimum(d_k, E_LIMIT), 16)

        @pl.when(n_chunks > 0)
        def _():
            pltpu.async_copy(eT_hbm.at[:, pl.ds(_cstart(0), CW)], ebuf0, esem0)

        @pl.when(n_chunks > 1)
        def _():
            pltpu.async_copy(eT_hbm.at[:, pl.ds(_cstart(1), CW)], ebuf1, esem1)

        def _pair(tp, accs):
            for half, (buf, sem) in enumerate(((ebuf0, esem0), (ebuf1, esem1))):
                t = 2 * tp + half
                d_k = ca + t * CW
                c_k = _cstart(t)

                @pl.when(t < n_chunks)
                def _():
                    pltpu.make_async_copy(
                        eT_hbm.at[:, pl.ds(c_k, CW)], buf, sem).wait()

                boff = d_k - c_k
                ng = jnp.maximum(
                    (jnp.minimum(d_k + CW, fl_b) - d_k) // 16, 0)
                nb = ng // UNROLL

                def _blk(i, accs):
                    base = boff + i * (UNROLL * 16)
                    new = list(accs)
                    for u in range(UNROLL):
                        for d in range(EDGE_DIM):
                            new[d] = new[d] + buf[d, pl.ds(base + u * 16, 16)]
                    return tuple(new)

                accs = lax.fori_loop(0, nb, _blk, accs)

                def _rem(q, accs):
                    off = boff + (nb * UNROLL + q) * 16
                    return tuple(accs[d] + buf[d, pl.ds(off, 16)]
                                 for d in range(EDGE_DIM))

                accs = lax.fori_loop(0, ng - nb * UNROLL, _rem, accs)

                @pl.when(t + 2 < n_chunks)
                def _():
                    pltpu.async_copy(
                        eT_hbm.at[:, pl.ds(_cstart(t + 2), CW)], buf, sem)
            return accs

        accs = lax.fori_loop(0, (n_chunks + 1) // 2, _pair,
                             tuple(zero16 for _ in range(EDGE_DIM)))

        for d in range(EDGE_DIM):
            obuf[gloc, d, pl.ds(0, 16)] = (
                accs[d]
                + jnp.where(head_mask, fbufh[d, pl.ds(0, 16)], 0.0)
                + jnp.where(tail_mask, fbuft[d, pl.ds(0, 16)], 0.0))
        return _

    lax.fori_loop(0, G_PER_TILE, _graph, None)
    pltpu.sync_copy(obuf,
                    ep_out.at[pl.ds(wid * G_PER_TILE, G_PER_TILE), :, :])


def _segsum_body(nodes_hbm, eT_hbm, nid_hbm, starts_hbm,
                 np_out, ep_out,
                 nbuf0, nbuf1, ebuf0, ebuf1, fbufh, fbuft, obuf, nidxb,
                 startsb, ntrows, ntidx, zrow, sem0, sem1, nacc):
    c = lax.axis_index("c")
    s = lax.axis_index("s")
    wid = c * 16 + s

    n_start = wid * N_CNT + jnp.minimum(wid, N_EXTRA)
    n_cnt = N_CNT + jnp.where(wid < N_EXTRA, 1, 0)

    pltpu.sync_copy(nid_hbm.at[pl.ds(n_start * L, N_MAX * L)], nidxb)
    pltpu.sync_copy(starts_hbm, startsb)

    def _zero_row(i, _):
        zrow[pl.ds(i * 16, 16)] = jnp.zeros((16,), jnp.float32)
        return _
    lax.fori_loop(0, 8, _zero_row, None)
    base = s * (N_GRAPHS // 16)

    def _zero_nacc(i, _):
        pltpu.sync_copy(zrow, nacc.at[base + i])
        return _
    lax.fori_loop(0, N_GRAPHS // 16, _zero_nacc, None)

    plsc.subcore_barrier()

    _node_phase(nodes_hbm, nidxb, nacc, nbuf0, nbuf1, sem0, sem1,
                n_start, n_cnt)

    @pl.when(wid == 30)
    def _():
        pltpu.sync_copy(nodes_hbm.at[pl.ds(N_FULL * L, N_TAIL), :], ntrows)
        pltpu.sync_copy(nid_hbm.at[pl.ds(N_FULL * L, N_TAIL)], ntidx)
        pltpu.sync_copy(ntrows, nacc.at[ntidx], add=True)

    _edge_phase(eT_hbm, startsb, ep_out, ebuf0, ebuf1, fbufh, fbuft, obuf,
                sem0, sem1, wid)

    plsc.subcore_barrier()

    rows = N_GRAPHS // 16
    pltpu.sync_copy(nacc.at[pl.ds(s * rows, rows), :],
                    np_out.at[c, pl.ds(s * rows, rows), :])


def _mlp_body(np_ref, ep_ref, gg_ref, w1a_ref, w1b_ref, w1c_ref, b1_ref,
              w2_ref, b2_ref, out_ref):
    ns = np_ref[0] + np_ref[1]
    es = jnp.sum(ep_ref[...], axis=2)
    x = (jnp.dot(gg_ref[...], w1a_ref[...], preferred_element_type=jnp.float32)
         + jnp.dot(ns, w1b_ref[...], preferred_element_type=jnp.float32)
         + jnp.dot(es, w1c_ref[...], preferred_element_type=jnp.float32)
         + b1_ref[...])
    h = jnp.maximum(x, 0.0)
    out_ref[...] = (jnp.dot(h, w2_ref[...], preferred_element_type=jnp.float32)
                    + b2_ref[...])


def kernel(nodes, batch, edges, batch_edges, graph_globals, W1, b1, W2, b2):
    bid = jnp.pad(batch.astype(jnp.int32), (0, N_MAX * L))
    eid = batch_edges.astype(jnp.int32)
    starts = jnp.searchsorted(eid, jnp.arange(N_GRAPHS + 1, dtype=jnp.int32),
                              side="left",
                              method="scan_unrolled").astype(jnp.int32)
    starts = jnp.pad(starts, (0, 7))
    edgesT = edges.T

    mesh = plsc.VectorSubcoreMesh(core_axis_name="c", subcore_axis_name="s")
    segsum = pl.kernel(
        _segsum_body,
        out_type=[
            jax.ShapeDtypeStruct((2, N_GRAPHS, NODE_DIM), jnp.float32),
            jax.ShapeDtypeStruct((N_GRAPHS, EDGE_DIM, 16), jnp.float32),
        ],
        mesh=mesh,
        compiler_params=pltpu.CompilerParams(use_tc_tiling_on_sc=False),
        scratch_types=[
            pltpu.VMEM((L, NODE_DIM), jnp.float32),
            pltpu.VMEM((L, NODE_DIM), jnp.float32),
            pltpu.VMEM((EDGE_DIM, CW), jnp.float32),
            pltpu.VMEM((EDGE_DIM, CW), jnp.float32),
            pltpu.VMEM((EDGE_DIM, 16), jnp.float32),
            pltpu.VMEM((EDGE_DIM, 16), jnp.float32),
            pltpu.VMEM((G_PER_TILE, EDGE_DIM, 16), jnp.float32),
            pltpu.VMEM((N_MAX * L,), jnp.int32),
            pltpu.VMEM((520,), jnp.int32),
            pltpu.VMEM((N_TAIL, NODE_DIM), jnp.float32),
            pltpu.VMEM((N_TAIL,), jnp.int32),
            pltpu.VMEM((NODE_DIM,), jnp.float32),
            pltpu.SemaphoreType.DMA,
            pltpu.SemaphoreType.DMA,
            pltpu.VMEM_SHARED((N_GRAPHS, NODE_DIM), jnp.float32),
        ],
    )
    np_part, ep_sum = segsum(nodes, edgesT, bid, starts)

    w1a = lax.slice(W1, (0, 0), (NODE_DIM, HIDDEN))
    w1b = lax.slice(W1, (NODE_DIM, 0), (2 * NODE_DIM, HIDDEN))
    w1c = lax.slice(W1, (2 * NODE_DIM, 0), (2 * NODE_DIM + EDGE_DIM, HIDDEN))

    out = pl.pallas_call(
        _mlp_body,
        out_shape=jax.ShapeDtypeStruct((N_GRAPHS, NODE_DIM), jnp.float32),
    )(np_part, ep_sum, graph_globals, w1a, w1b, w1c,
      b1.reshape(1, HIDDEN), W2, b2.reshape(1, NODE_DIM))
    return out


# speedup vs baseline: 1.0840x; 1.0840x over previous
import jax
import jax.numpy as jnp
from jax import lax
from jax.experimental import pallas as pl
from jax.experimental.pallas import tpu as pltpu
from jax.experimental.pallas import tpu_sc as plsc

N_GRAPHS = 512
N_NODES = 100000
N_EDGES = 1600000
NODE_DIM = 128
EDGE_DIM = 16
HIDDEN = 64

NW = 32
L = 128

N_FULL = N_NODES // L
N_TAIL = N_NODES - N_FULL * L
N_CNT = N_FULL // NW
N_EXTRA = N_FULL - N_CNT * NW
N_MAX = N_CNT + 1

CW = 2048
E_LIMIT = N_EDGES - CW
G_PER_TILE = N_GRAPHS // NW


def _node_phase(src_hbm, idxb, acc, buf0, buf1, sem0, sem1, start, cnt):
    @pl.when(cnt > 0)
    def _():
        pltpu.async_copy(src_hbm.at[pl.ds(start * L, L), :], buf0, sem0)

    @pl.when(cnt > 1)
    def _():
        pltpu.async_copy(src_hbm.at[pl.ds((start + 1) * L, L), :], buf1, sem1)

    def _pair(tp, _):
        for half, (buf, sem) in enumerate(((buf0, sem0), (buf1, sem1))):
            t = 2 * tp + half

            @pl.when(t < cnt)
            def _():
                pltpu.make_async_copy(
                    src_hbm.at[pl.ds((start + t) * L, L), :], buf, sem).wait()
                pltpu.sync_copy(buf, acc.at[idxb.at[pl.ds(t * L, L)]],
                                add=True)

                @pl.when(t + 2 < cnt)
                def _():
                    pltpu.async_copy(
                        src_hbm.at[pl.ds((start + t + 2) * L, L), :], buf, sem)
        return _

    lax.fori_loop(0, (N_MAX + 1) // 2, _pair, None)


def _edge_phase(eT_hbm, starts, ep_out, ebuf0, ebuf1, fbufh, fbuft, obuf,
                esem0, esem1, wid):
    iota = lax.iota(jnp.int32, 16)
    zero16 = jnp.zeros((16,), jnp.float32)
    UNROLL = 8

    def _graph(gloc, _):
        g = wid * G_PER_TILE + gloc
        win = starts[pl.ds(g, 16)]
        a = win[0]
        b = win[1]
        fl_a = pl.multiple_of((a // 16) * 16, 16)
        fl_b = pl.multiple_of((b // 16) * 16, 16)
        ca = pl.multiple_of(jnp.where(a == fl_a, a, fl_a + 16), 16)

        @pl.when(a < b)
        def _():
            pltpu.sync_copy(eT_hbm.at[:, pl.ds(fl_a, 16)], fbufh)

        @pl.when((fl_b > fl_a) & (b != fl_b))
        def _():
            pltpu.sync_copy(eT_hbm.at[:, pl.ds(fl_b, 16)], fbuft)

        head_mask = (iota >= a - fl_a) & (iota < jnp.minimum(ca, b) - fl_a)
        tail_mask = (iota < jnp.where(fl_b > fl_a, b - fl_b, 0))

        nfull = jnp.maximum(fl_b - ca, 0)
        n_chunks = (nfull + CW - 1) // CW

        def _cstart(t):
            d_k = ca + t * CW
            return pl.multiple_of(jnp.minimum(d_k, E_LIMIT), 16)

        @pl.when(n_chunks > 0)
        def _():
            pltpu.async_copy(eT_hbm.at[:, pl.ds(_cstart(0), CW)], ebuf0, esem0)

        @pl.when(n_chunks > 1)
        def _():
            pltpu.async_copy(eT_hbm.at[:, pl.ds(_cstart(1), CW)], ebuf1, esem1)

        def _pair(tp, accs):
            for half, (buf, sem) in enumerate(((ebuf0, esem0), (ebuf1, esem1))):
                t = 2 * tp + half
                d_k = ca + t * CW
                c_k = _cstart(t)

                @pl.when(t < n_chunks)
                def _():
                    pltpu.make_async_copy(
                        eT_hbm.at[:, pl.ds(c_k, CW)], buf, sem).wait()

                boff = d_k - c_k
                ng = jnp.maximum(
                    (jnp.minimum(d_k + CW, fl_b) - d_k) // 16, 0)
                nb = ng // UNROLL

                def _blk(i, accs):
                    base = boff + i * (UNROLL * 16)
                    new = list(accs)
                    for u in range(UNROLL):
                        for d in range(EDGE_DIM):
                            new[d] = new[d] + buf[d, pl.ds(base + u * 16, 16)]
                    return tuple(new)

                accs = lax.fori_loop(0, nb, _blk, accs)

                def _rem(q, accs):
                    off = boff + (nb * UNROLL + q) * 16
                    return tuple(accs[d] + buf[d, pl.ds(off, 16)]
                                 for d in range(EDGE_DIM))

                accs = lax.fori_loop(0, ng - nb * UNROLL, _rem, accs)

                @pl.when(t + 2 < n_chunks)
                def _():
                    pltpu.async_copy(
                        eT_hbm.at[:, pl.ds(_cstart(t + 2), CW)], buf, sem)
            return accs

        accs = lax.fori_loop(0, (n_chunks + 1) // 2, _pair,
                             tuple(zero16 for _ in range(EDGE_DIM)))

        for d in range(EDGE_DIM):
            obuf[gloc, d, pl.ds(0, 16)] = (
                accs[d]
                + jnp.where(head_mask, fbufh[d, pl.ds(0, 16)], 0.0)
                + jnp.where(tail_mask, fbuft[d, pl.ds(0, 16)], 0.0))
        return _

    lax.fori_loop(0, G_PER_TILE, _graph, None)
    pltpu.sync_copy(obuf,
                    ep_out.at[pl.ds(wid * G_PER_TILE, G_PER_TILE), :, :])


def _segsum_body(nodes_hbm, eT_hbm, nid_hbm, starts_hbm,
                 np_out, ep_out,
                 nbuf0, nbuf1, ebuf0, ebuf1, fbufh, fbuft, obuf, nidxb,
                 startsb, ntrows, ntidx, zrow, sem0, sem1, nacc):
    c = lax.axis_index("c")
    s = lax.axis_index("s")
    wid = c * 16 + s

    n_start = wid * N_CNT + jnp.minimum(wid, N_EXTRA)
    n_cnt = N_CNT + jnp.where(wid < N_EXTRA, 1, 0)

    pltpu.sync_copy(nid_hbm.at[pl.ds(n_start * L, N_MAX * L)], nidxb)
    pltpu.sync_copy(starts_hbm, startsb)

    def _zero_row(i, _):
        zrow[pl.ds(i * 16, 16)] = jnp.zeros((16,), jnp.float32)
        return _
    lax.fori_loop(0, 8, _zero_row, None)
    base = s * (N_GRAPHS // 16)

    def _zero_nacc(i, _):
        pltpu.sync_copy(zrow, nacc.at[base + i])
        return _
    lax.fori_loop(0, N_GRAPHS // 16, _zero_nacc, None)

    plsc.subcore_barrier()

    _node_phase(nodes_hbm, nidxb, nacc, nbuf0, nbuf1, sem0, sem1,
                n_start, n_cnt)

    @pl.when(wid == 30)
    def _():
        pltpu.sync_copy(nodes_hbm.at[pl.ds(N_FULL * L, N_TAIL), :], ntrows)
        pltpu.sync_copy(nid_hbm.at[pl.ds(N_FULL * L, N_TAIL)], ntidx)
        pltpu.sync_copy(ntrows, nacc.at[ntidx], add=True)

    _edge_phase(eT_hbm, startsb, ep_out, ebuf0, ebuf1, fbufh, fbuft, obuf,
                sem0, sem1, wid)

    plsc.subcore_barrier()

    rows = N_GRAPHS // 16
    pltpu.sync_copy(nacc.at[pl.ds(s * rows, rows), :],
                    np_out.at[c, pl.ds(s * rows, rows), :])


def _mlp_body(np_ref, ep_ref, gg_ref, w1a_ref, w1b_ref, w1c_ref, b1_ref,
              w2_ref, b2_ref, out_ref):
    ns = np_ref[0] + np_ref[1]
    es = jnp.sum(ep_ref[...], axis=2)
    x = (jnp.dot(gg_ref[...], w1a_ref[...], preferred_element_type=jnp.float32)
         + jnp.dot(ns, w1b_ref[...], preferred_element_type=jnp.float32)
         + jnp.dot(es, w1c_ref[...], preferred_element_type=jnp.float32)
         + b1_ref[...])
    h = jnp.maximum(x, 0.0)
    out_ref[...] = (jnp.dot(h, w2_ref[...], preferred_element_type=jnp.float32)
                    + b2_ref[...])


def kernel(nodes, batch, edges, batch_edges, graph_globals, W1, b1, W2, b2):
    bid = jnp.pad(batch.astype(jnp.int32), (0, N_MAX * L))
    eid = batch_edges.astype(jnp.int32)
    counts = jnp.zeros((N_GRAPHS,), jnp.int32).at[eid].add(1)
    starts = jnp.concatenate(
        [jnp.zeros((1,), jnp.int32), jnp.cumsum(counts)]).astype(jnp.int32)
    starts = jnp.pad(starts, (0, 7))
    edgesT = edges.T

    mesh = plsc.VectorSubcoreMesh(core_axis_name="c", subcore_axis_name="s")
    segsum = pl.kernel(
        _segsum_body,
        out_type=[
            jax.ShapeDtypeStruct((2, N_GRAPHS, NODE_DIM), jnp.float32),
            jax.ShapeDtypeStruct((N_GRAPHS, EDGE_DIM, 16), jnp.float32),
        ],
        mesh=mesh,
        compiler_params=pltpu.CompilerParams(use_tc_tiling_on_sc=False),
        scratch_types=[
            pltpu.VMEM((L, NODE_DIM), jnp.float32),
            pltpu.VMEM((L, NODE_DIM), jnp.float32),
            pltpu.VMEM((EDGE_DIM, CW), jnp.float32),
            pltpu.VMEM((EDGE_DIM, CW), jnp.float32),
            pltpu.VMEM((EDGE_DIM, 16), jnp.float32),
            pltpu.VMEM((EDGE_DIM, 16), jnp.float32),
            pltpu.VMEM((G_PER_TILE, EDGE_DIM, 16), jnp.float32),
            pltpu.VMEM((N_MAX * L,), jnp.int32),
            pltpu.VMEM((520,), jnp.int32),
            pltpu.VMEM((N_TAIL, NODE_DIM), jnp.float32),
            pltpu.VMEM((N_TAIL,), jnp.int32),
            pltpu.VMEM((NODE_DIM,), jnp.float32),
            pltpu.SemaphoreType.DMA,
            pltpu.SemaphoreType.DMA,
            pltpu.VMEM_SHARED((N_GRAPHS, NODE_DIM), jnp.float32),
        ],
    )
    np_part, ep_sum = segsum(nodes, edgesT, bid, starts)

    w1a = lax.slice(W1, (0, 0), (NODE_DIM, HIDDEN))
    w1b = lax.slice(W1, (NODE_DIM, 0), (2 * NODE_DIM, HIDDEN))
    w1c = lax.slice(W1, (2 * NODE_DIM, 0), (2 * NODE_DIM + EDGE_DIM, HIDDEN))

    out = pl.pallas_call(
        _mlp_body,
        out_shape=jax.ShapeDtypeStruct((N_GRAPHS, NODE_DIM), jnp.float32),
    )(np_part, ep_sum, graph_globals, w1a, w1b, w1c,
      b1.reshape(1, HIDDEN), W2, b2.reshape(1, NODE_DIM))
    return out
